# baseline (speedup 1.0000x reference)
import jax
import jax.numpy as jnp
from jax import lax
from jax.experimental import pallas as pl
from jax.experimental.pallas import tpu as pltpu
from jax.experimental.pallas import tpu_sc as plsc

VOCAB = 1000000
EMBED_DIM = 32
BATCH = 16384
HIST = 200

NUM_CORES = 2
NUM_SUBCORES = 16
NW = NUM_CORES * NUM_SUBCORES

BB = 128
NBLK = BATCH // BB
BLK_PER_W = NBLK // NW
HT = HIST // 8
DT = EMBED_DIM // 8

TB = 16384
VPAD = ((VOCAB + TB - 1) // TB) * TB

GH = 4
NBUF = 5
GROUPS = HIST // GH
ITERS = GROUPS // NBUF


def _sc_gather_body(idx_hbm, table_hbm, out_hbm, idxv, rows, drainb,
                    sem_i, *sems):
    wid = lax.axis_index("s") * NUM_CORES + lax.axis_index("c")
    gsem = sems[:NBUF]
    ssem = sems[NBUF:]

    def issue_gathers(g, buf):
        for hh in range(GH):
            h = g * GH + hh
            pltpu.async_copy(
                table_hbm.at[idxv.at[h // 8, h % 8]],
                rows.at[buf, hh],
                gsem[buf],
            )


    def drain_gathers(buf):
        for hh in range(GH):
            pltpu.make_async_copy(
                table_hbm.at[pl.ds(0, BB)], rows.at[buf, hh], gsem[buf]
            ).wait()

    def issue_stores(g, buf, bb):
        for hh in range(GH):
            for bt8 in range(16):
                pltpu.async_copy(
                    rows.at[buf, hh, pl.ds(bt8 * 8, 8)],
                    out_hbm.at[bb * 16 + bt8, g, :, pl.ds(hh * EMBED_DIM,
                                                          EMBED_DIM)],
                    ssem[buf],
                )

    def drain_stores(buf):
        for hh in range(GH):
            pltpu.make_async_copy(
                out_hbm.at[pl.ds(0, 16), 0, :, pl.ds(0, EMBED_DIM)],
                drainb, ssem[buf],
            ).wait()

    def run_block(b, carry):
        bb = wid * BLK_PER_W + b
        pltpu.async_copy(idx_hbm.at[:, bb], idxv, sem_i).wait()
        issue_gathers(0, 0)

        def iteration(t, c):
            for k in range(NBUF):
                g = t * NBUF + k
                buf = k
                nbuf = (k + 1) % NBUF
                drain_gathers(buf)

                @pl.when((g < GROUPS - 1) & (g >= NBUF - 1))
                def _():
                    drain_stores(nbuf)

                @pl.when(g < GROUPS - 1)
                def _():
                    issue_gathers(g + 1, nbuf)

                issue_stores(g, buf, bb)
            return c

        lax.fori_loop(0, ITERS, iteration, 0)
        for buf in range(NBUF):
            drain_stores(buf)
        return carry

    lax.fori_loop(0, BLK_PER_W, run_block, 0)


@jax.jit
def _embed_lookup(idx4, table):
    mesh = plsc.VectorSubcoreMesh(
        core_axis_name="c", subcore_axis_name="s",
        num_cores=NUM_CORES, num_subcores=NUM_SUBCORES,
    )
    sc_fn = pl.kernel(
        _sc_gather_body,
        out_type=jax.ShapeDtypeStruct(
            (BATCH // 8, HIST * EMBED_DIM // BB, 8, BB), jnp.float32),
        mesh=mesh,
        scratch_types=(
            [
                pltpu.VMEM((HT, 8, BB), jnp.int32),
                pltpu.VMEM((NBUF, GH, BB, EMBED_DIM), jnp.float32),
                pltpu.VMEM((16, 8, EMBED_DIM), jnp.float32),
            ]
            + [pltpu.SemaphoreType.DMA] * (1 + 2 * NBUF)
        ),
        compiler_params=pltpu.CompilerParams(use_tc_tiling_on_sc=False),
    )
    def tc_detile(in_ref, out_ref):
        y = jax.lax.dot_general(
            in_ref[...], jnp.eye(EMBED_DIM, dtype=jnp.float32),
            (((0,), (0,)), ((), ())),
            preferred_element_type=jnp.float32)
        y3 = y.reshape(TB // 4, 4, EMBED_DIM)
        for q in range(4):
            out_ref[:, q * EMBED_DIM:(q + 1) * EMBED_DIM] = y3[:, q, :]

    tab_lin = pl.pallas_call(
        tc_detile,
        grid=(VPAD // TB,),
        in_specs=[pl.BlockSpec((EMBED_DIM, TB), lambda i: (0, i))],
        out_specs=pl.BlockSpec((TB // 4, BB), lambda i: (i, 0)),
        out_shape=jax.ShapeDtypeStruct((VPAD * EMBED_DIM // BB, BB),
                                       jnp.float32),
    )(table.T)
    tab_sc = tab_lin.reshape(VPAD, EMBED_DIM)

    out4 = sc_fn(idx4, tab_sc)
    m = out4.transpose(0, 2, 1, 3).reshape(BATCH, HIST * EMBED_DIM)

    def tc_tilize(in_ref, out_ref):
        x = in_ref[...]
        y = x.T
        out_ref[...] = y.reshape(HIST, DT, 1, 8, BB)

    out5 = pl.pallas_call(
        tc_tilize,
        grid=(NBLK,),
        in_specs=[pl.BlockSpec((BB, HIST * EMBED_DIM), lambda i: (i, 0))],
        out_specs=pl.BlockSpec((HIST, DT, 1, 8, BB),
                               lambda i: (0, 0, i, 0, 0)),
        out_shape=jax.ShapeDtypeStruct((HIST, DT, NBLK, 8, BB), jnp.float32),
    )(m)
    return out5


def kernel(input_sequence, embedding_weight):
    idx4 = (
        input_sequence.astype(jnp.int32)
        .reshape(NBLK, BB, HT, 8)
        .transpose(2, 0, 3, 1)
    )
    out5 = _embed_lookup(idx4, embedding_weight)
    return out5.transpose(2, 4, 0, 1, 3).reshape(BATCH, HIST, EMBED_DIM)


# speedup vs baseline: 1.0216x; 1.0216x over previous
import jax
import jax.numpy as jnp
from jax import lax
from jax.experimental import pallas as pl
from jax.experimental.pallas import tpu as pltpu
from jax.experimental.pallas import tpu_sc as plsc

VOCAB = 1000000
EMBED_DIM = 32
BATCH = 16384
HIST = 200

NUM_CORES = 2
NUM_SUBCORES = 16
NW = NUM_CORES * NUM_SUBCORES

BB = 128
NBLK = BATCH // BB
BLK_PER_W = NBLK // NW
HT = HIST // 8
DT = EMBED_DIM // 8

TB = 16384
VPAD = ((VOCAB + TB - 1) // TB) * TB

GH = 4
NBUF = 5
GROUPS = HIST // GH
ITERS = GROUPS // NBUF


def _sc_gather_body(idx_hbm, table_hbm, out_hbm, idxv, rows, drainb,
                    sem_i, *sems):
    wid = lax.axis_index("s") * NUM_CORES + lax.axis_index("c")
    gsem = sems[:NBUF]
    ssem = sems[NBUF:]

    def issue_gathers(g, buf):
        for hh in range(GH):
            h = g * GH + hh
            pltpu.async_copy(
                table_hbm.at[idxv.at[h // 8, h % 8]],
                rows.at[buf, hh],
                gsem[buf],
            )


    def drain_gathers(buf):
        for hh in range(GH):
            pltpu.make_async_copy(
                table_hbm.at[pl.ds(0, BB)], rows.at[buf, hh], gsem[buf]
            ).wait()

    def issue_stores(g, buf, bb):
        for hh in range(GH):
            for bt8 in range(16):
                pltpu.async_copy(
                    rows.at[buf, hh, pl.ds(bt8 * 8, 8)],
                    out_hbm.at[bb * 16 + bt8, g, :, pl.ds(hh * EMBED_DIM,
                                                          EMBED_DIM)],
                    ssem[buf],
                )

    def drain_stores(buf):
        for hh in range(GH):
            pltpu.make_async_copy(
                out_hbm.at[pl.ds(0, 16), 0, :, pl.ds(0, EMBED_DIM)],
                drainb, ssem[buf],
            ).wait()

    def run_block(b, carry):
        bb = wid * BLK_PER_W + b
        pltpu.async_copy(idx_hbm.at[:, bb], idxv, sem_i).wait()
        issue_gathers(0, 0)

        def iteration(t, c):
            for k in range(NBUF):
                g = t * NBUF + k
                buf = k
                nbuf = (k + 1) % NBUF
                drain_gathers(buf)

                @pl.when((g < GROUPS - 1) & (g >= NBUF - 1))
                def _():
                    drain_stores(nbuf)

                @pl.when(g < GROUPS - 1)
                def _():
                    issue_gathers(g + 1, nbuf)

                issue_stores(g, buf, bb)
            return c

        lax.fori_loop(0, ITERS, iteration, 0)
        for buf in range(NBUF):
            drain_stores(buf)
        return carry

    lax.fori_loop(0, BLK_PER_W, run_block, 0)


@jax.jit
def _embed_lookup(idx4, table):
    mesh = plsc.VectorSubcoreMesh(
        core_axis_name="c", subcore_axis_name="s",
        num_cores=NUM_CORES, num_subcores=NUM_SUBCORES,
    )
    sc_fn = pl.kernel(
        _sc_gather_body,
        out_type=jax.ShapeDtypeStruct(
            (BATCH // 8, HIST * EMBED_DIM // BB, 8, BB), jnp.float32),
        mesh=mesh,
        scratch_types=(
            [
                pltpu.VMEM((HT, 8, BB), jnp.int32),
                pltpu.VMEM((NBUF, GH, BB, EMBED_DIM), jnp.float32),
                pltpu.VMEM((16, 8, EMBED_DIM), jnp.float32),
            ]
            + [pltpu.SemaphoreType.DMA] * (1 + 2 * NBUF)
        ),
        compiler_params=pltpu.CompilerParams(use_tc_tiling_on_sc=False),
    )
    def tc_detile(in_ref, out_ref):
        y = in_ref[...].T
        y3 = y.reshape(TB // 4, 4, EMBED_DIM)
        for q in range(4):
            out_ref[:, q * EMBED_DIM:(q + 1) * EMBED_DIM] = y3[:, q, :]

    tab_lin = pl.pallas_call(
        tc_detile,
        grid=(VPAD // TB,),
        in_specs=[pl.BlockSpec((EMBED_DIM, TB), lambda i: (0, i))],
        out_specs=pl.BlockSpec((TB // 4, BB), lambda i: (i, 0)),
        out_shape=jax.ShapeDtypeStruct((VPAD * EMBED_DIM // BB, BB),
                                       jnp.float32),
    )(table.T)
    tab_sc = tab_lin.reshape(VPAD, EMBED_DIM)

    out4 = sc_fn(idx4, tab_sc)
    m = out4.transpose(0, 2, 1, 3).reshape(BATCH, HIST * EMBED_DIM)

    def tc_tilize(in_ref, out_ref):
        x = in_ref[...]
        y = x.T
        out_ref[...] = y.reshape(HIST, DT, 1, 8, BB)

    out5 = pl.pallas_call(
        tc_tilize,
        grid=(NBLK,),
        in_specs=[pl.BlockSpec((BB, HIST * EMBED_DIM), lambda i: (i, 0))],
        out_specs=pl.BlockSpec((HIST, DT, 1, 8, BB),
                               lambda i: (0, 0, i, 0, 0)),
        out_shape=jax.ShapeDtypeStruct((HIST, DT, NBLK, 8, BB), jnp.float32),
    )(m)
    return out5


def kernel(input_sequence, embedding_weight):
    idx4 = (
        input_sequence.astype(jnp.int32)
        .reshape(NBLK, BB, HT, 8)
        .transpose(2, 0, 3, 1)
    )
    out5 = _embed_lookup(idx4, embedding_weight)
    return out5.transpose(2, 4, 0, 1, 3).reshape(BATCH, HIST, EMBED_DIM)
